# baseline (speedup 1.0000x reference)
import functools

import jax
import jax.numpy as jnp
from jax import lax
from jax.experimental import pallas as pl
from jax.experimental.pallas import tpu as pltpu
from jax.experimental.pallas import tpu_sc as plsc

N = 10000
E = 320000
HID = 128
NH = 8
HD = HID // NH
EPS = 1e-5

NC = 2
NS = 16
NW = NC * NS
EW = E // NW
C = 16
NCHUNK = EW // C
NP = 10240
NP8 = NP // 8
AROWS = NP + NP8
RPT = AROWS // NS
NB = NP // NW

_GDN = lax.GatherDimensionNumbers(
    offset_dims=(), collapsed_slice_dims=(0,), start_index_map=(0,))


def _perm(x, idx):
    return lax.gather(x, idx[:, None], _GDN, (1,),
                      mode=lax.GatherScatterMode.PROMISE_IN_BOUNDS)


def _qkv_body(h_ref, wq_ref, bq_ref, wk_ref, bk_ref, wv_ref, bv_ref,
              qs_ref, k_ref, v_ref):
    h = h_ref[...]
    scaling = HD ** (-0.5)
    qs_ref[...] = (jnp.dot(h, wq_ref[...],
                           preferred_element_type=jnp.float32)
                   + bq_ref[...]) * scaling
    k_ref[...] = jnp.dot(h, wk_ref[...],
                         preferred_element_type=jnp.float32) + bk_ref[...]
    v_ref[...] = jnp.dot(h, wv_ref[...],
                         preferred_element_type=jnp.float32) + bv_ref[...]


_qkv_call = pl.pallas_call(
    _qkv_body,
    out_shape=[
        jax.ShapeDtypeStruct((N, HID), jnp.float32),
        jax.ShapeDtypeStruct((N, HID), jnp.float32),
        jax.ShapeDtypeStruct((N, HID), jnp.float32),
    ],
)


def _sc_body(rows_hbm, cols_hbm, qs_hbm, k_hbm, v_hbm,
             out_hbm,
             acc,
             rows_a, cols_a, rows_b, cols_b,
             qb_a, kb_a, vb_a, ev_a, qb_b, kb_b, vb_b, ev_b, zq,
             sq_a, sk_a, sv_a, sm_a, se_a,
             sq_b, sk_b, sv_b, sm_b, se_b):
    c = lax.axis_index("c")
    s = lax.axis_index("s")
    wid = s * NC + c
    base = wid * EW
    lane = lax.iota(jnp.int32, 16)
    idx_hi = (lane + 8) & 15
    idx_mod8 = lane & 7
    zero16 = jnp.zeros((16,), jnp.float32)

    def _zq_row(i, _):
        for j in range(8):
            zq[i, pl.ds(16 * j, 16)] = zero16
        return 0
    lax.fori_loop(0, 16, _zq_row, 0)

    r0 = s * RPT

    def _zcopy(t, _):
        pltpu.sync_copy(zq, acc.at[pl.ds(r0 + t * 16, 16)])
        return 0
    lax.fori_loop(0, RPT // 16, _zcopy, 0)

    plsc.subcore_barrier()

    DEN_BATCHES = ((0, 0),)

    def _load_idx(ch, rv, cv):
        off = base + ch * C
        pltpu.sync_copy(rows_hbm.at[pl.ds(off, C)], rv)
        pltpu.sync_copy(cols_hbm.at[pl.ds(off, C)], cv)

    def _start_gathers(rv, cv, qb, kb, vb, sq, sk, sv):
        rr = rv[pl.ds(0, 16)]
        cc = cv[pl.ds(0, 16)]
        pltpu.async_copy(qs_hbm.at[rr], qb, sq)
        pltpu.async_copy(k_hbm.at[cc], kb, sk)
        pltpu.async_copy(v_hbm.at[cc], vb, sv)

    def _wait_gathers(rv, cv, qb, kb, vb, sq, sk, sv):
        pltpu.make_async_copy(qs_hbm.at[pl.ds(0, 16)], qb, sq).wait()
        pltpu.make_async_copy(k_hbm.at[pl.ds(0, 16)], kb, sk).wait()
        pltpu.make_async_copy(v_hbm.at[pl.ds(0, 16)], vb, sv).wait()

    def _compute(rv, qb, kb, vb, ev):
        k7 = (rv[pl.ds(0, 16)] & 7).astype(jnp.float32)

        def _edge(e, _):
            dot = qb[e, pl.ds(0, 16)] * kb[e, pl.ds(0, 16)]
            for j in range(1, 8):
                dot = dot + qb[e, pl.ds(16 * j, 16)] * kb[e, pl.ds(16 * j, 16)]
            s16 = dot + _perm(dot, idx_hi)
            s16 = jnp.where(lane < 8, s16, 0.0)
            e16 = jnp.exp(s16)
            e16z = jnp.where(lane < 8, e16, 0.0)
            ks = _perm(k7, lane * 0 + e)
            for j in range(8):
                d = ks - float(j)
                ev[e, pl.ds(16 * j, 16)] = e16z * jnp.maximum(1.0 - d * d, 0.0)
            pat = _perm(e16, idx_mod8)
            for j in range(8):
                vb[e, pl.ds(16 * j, 16)] = vb[e, pl.ds(16 * j, 16)] * pat
            return 0

        lax.fori_loop(0, C, _edge, 0)

    def _start_scatters(rv, vb, ev, sm, se):
        rr = rv[pl.ds(0, 16)]
        di = NP + lax.shift_right_logical(rr, 3)
        pltpu.async_copy(vb, acc.at[rr], sm, add=True)
        pltpu.async_copy(ev, acc.at[di], se, add=True)

    def _drain_scatters(rv, vb, ev, sm, se):
        pltpu.make_async_copy(vb, acc.at[pl.ds(0, 16)], sm).wait()
        pltpu.make_async_copy(ev, acc.at[pl.ds(0, 16)], se).wait()

    _load_idx(0, rows_a, cols_a)
    _start_gathers(rows_a, cols_a, qb_a, kb_a, vb_a, sq_a, sk_a, sv_a)

    def _pair(i, _):
        @pl.when(i > 0)
        def _():
            _drain_scatters(rows_b, vb_b, ev_b, sm_b, se_b)
        _load_idx(2 * i + 1, rows_b, cols_b)
        _start_gathers(rows_b, cols_b, qb_b, kb_b, vb_b, sq_b, sk_b, sv_b)
        _wait_gathers(rows_a, cols_a, qb_a, kb_a, vb_a, sq_a, sk_a, sv_a)
        _compute(rows_a, qb_a, kb_a, vb_a, ev_a)
        _start_scatters(rows_a, vb_a, ev_a, sm_a, se_a)
        _drain_scatters(rows_a, vb_a, ev_a, sm_a, se_a)
        _load_idx(2 * i + 2, rows_a, cols_a)
        _start_gathers(rows_a, cols_a, qb_a, kb_a, vb_a, sq_a, sk_a, sv_a)
        _wait_gathers(rows_b, cols_b, qb_b, kb_b, vb_b, sq_b, sk_b, sv_b)
        _compute(rows_b, qb_b, kb_b, vb_b, ev_b)
        _start_scatters(rows_b, vb_b, ev_b, sm_b, se_b)
        return 0

    lax.fori_loop(0, NCHUNK // 2, _pair, 0)
    _drain_scatters(rows_b, vb_b, ev_b, sm_b, se_b)
    _wait_gathers(rows_a, cols_a, qb_a, kb_a, vb_a, sq_a, sk_a, sv_a)
    _compute(rows_a, qb_a, kb_a, vb_a, ev_a)
    _start_scatters(rows_a, vb_a, ev_a, sm_a, se_a)
    _drain_scatters(rows_a, vb_a, ev_a, sm_a, se_a)
    plsc.subcore_barrier()

    def _wcopy(t, _):
        pltpu.sync_copy(acc.at[pl.ds(r0 + t * 16, 16)],
                        out_hbm.at[c, pl.ds(r0 + t * 16, 16)])
        return 0
    lax.fori_loop(0, RPT // 16, _wcopy, 0)


_sc_call = functools.partial(
    pl.kernel,
    out_type=jax.ShapeDtypeStruct((NC, AROWS, HID), jnp.float32),
    mesh=plsc.VectorSubcoreMesh(
        core_axis_name="c", subcore_axis_name="s",
        num_cores=NC, num_subcores=NS),
    scratch_types=[
        pltpu.VMEM_SHARED((AROWS, HID), jnp.float32),
        pltpu.VMEM((C,), jnp.int32),
        pltpu.VMEM((C,), jnp.int32),
        pltpu.VMEM((C,), jnp.int32),
        pltpu.VMEM((C,), jnp.int32),
        pltpu.VMEM((C, HID), jnp.float32),
        pltpu.VMEM((C, HID), jnp.float32),
        pltpu.VMEM((C, HID), jnp.float32),
        pltpu.VMEM((C, HID), jnp.float32),
        pltpu.VMEM((C, HID), jnp.float32),
        pltpu.VMEM((C, HID), jnp.float32),
        pltpu.VMEM((C, HID), jnp.float32),
        pltpu.VMEM((C, HID), jnp.float32),
        pltpu.VMEM((16, HID), jnp.float32),
    ] + [pltpu.SemaphoreType.DMA] * 10,
)(_sc_body)


PC = 64


def _norm_body(all_hbm,
               att_hbm,
               o0, o1, d0, d1, obuf, sem0, sem1, sem2, sem3):
    c = lax.axis_index("c")
    s = lax.axis_index("s")
    wid = s * NC + c
    nb = wid * NB
    lane = lax.iota(jnp.int32, 16)
    idx_mod8 = lane & 7

    def _chunk(t, _):
        nb_t = pl.multiple_of(nb + t * PC, 64)
        dof = pl.multiple_of(NP + wid * (NB // 8) + t * (PC // 8), 8)
        cp0 = pltpu.async_copy(all_hbm.at[0, pl.ds(nb_t, PC)], o0, sem0)
        cp1 = pltpu.async_copy(all_hbm.at[1, pl.ds(nb_t, PC)], o1, sem1)
        cp2 = pltpu.async_copy(all_hbm.at[0, pl.ds(dof, PC // 8)], d0, sem2)
        cp3 = pltpu.async_copy(all_hbm.at[1, pl.ds(dof, PC // 8)], d1, sem3)
        cp0.wait()
        cp1.wait()
        cp2.wait()
        cp3.wait()

        def _drow(dr, _):
            for j in range(8):
                slot = d0[dr, pl.ds(16 * j, 16)] + d1[dr, pl.ds(16 * j, 16)]
                recip = 1.0 / jnp.maximum(slot, 1e-30)
                pat = _perm(recip, idx_mod8)
                i = dr * 8 + j
                for m in range(8):
                    obuf[i, pl.ds(16 * m, 16)] = (
                        o0[i, pl.ds(16 * m, 16)]
                        + o1[i, pl.ds(16 * m, 16)]) * pat
            return 0

        lax.fori_loop(0, PC // 8, _drow, 0)
        pltpu.sync_copy(obuf, att_hbm.at[pl.ds(nb_t, PC)])
        return 0

    lax.fori_loop(0, NB // PC, _chunk, 0)


_norm_call = functools.partial(
    pl.kernel,
    out_type=jax.ShapeDtypeStruct((NP, HID), jnp.float32),
    mesh=plsc.VectorSubcoreMesh(
        core_axis_name="c", subcore_axis_name="s",
        num_cores=NC, num_subcores=NS),
    scratch_types=[
        pltpu.VMEM((PC, HID), jnp.float32),
        pltpu.VMEM((PC, HID), jnp.float32),
        pltpu.VMEM((PC // 8, HID), jnp.float32),
        pltpu.VMEM((PC // 8, HID), jnp.float32),
        pltpu.VMEM((PC, HID), jnp.float32),
        pltpu.SemaphoreType.DMA,
        pltpu.SemaphoreType.DMA,
        pltpu.SemaphoreType.DMA,
        pltpu.SemaphoreType.DMA,
    ],
)(_norm_body)


def _gelu(x):
    return 0.5 * x * (1.0 + lax.erf(x * (2.0 ** -0.5)))


def _bn(x, gamma, beta):
    mean = jnp.mean(x, axis=0)
    var = jnp.mean((x - mean) ** 2, axis=0)
    return (x - mean) / jnp.sqrt(var + EPS) * gamma + beta


def _epi_body(a_ref, h_ref, wo_ref, bo_ref, w1_ref, b1_ref,
              w2_ref, b2_ref, g1_ref, be1_ref, g2_ref, be2_ref, out_ref):
    attnout = a_ref[:N]
    mha = jnp.dot(attnout, wo_ref[...],
                  preferred_element_type=jnp.float32) + bo_ref[...]
    x1 = _bn(mha + h_ref[...], g1_ref[...], be1_ref[...])
    h1 = _gelu(x1)
    ffn = _gelu(jnp.dot(h1, w1_ref[...],
                        preferred_element_type=jnp.float32) + b1_ref[...])
    ffn = jnp.dot(ffn, w2_ref[...],
                  preferred_element_type=jnp.float32) + b2_ref[...]
    out_ref[...] = _bn(h1 + ffn, g2_ref[...], be2_ref[...])


_epi_call = pl.pallas_call(
    _epi_body,
    out_shape=jax.ShapeDtypeStruct((N, HID), jnp.float32),
)


def kernel(edge_index, h, Wq, bq, Wk, bk, Wv, bv, Wo, bo, W1, b1, W2, b2,
           g1, be1, g2, be2):
    rows = edge_index[0]
    cols = edge_index[1]
    qs, kk, vv = _qkv_call(h, Wq, bq.reshape(1, HID), Wk, bk.reshape(1, HID),
                           Wv, bv.reshape(1, HID))
    out_all = _sc_call(rows, cols, qs, kk, vv)
    att = _norm_call(out_all)
    return _epi_call(att, h, Wo, bo.reshape(1, HID),
                     W1, b1.reshape(1, 2 * HID), W2, b2.reshape(1, HID),
                     g1.reshape(1, HID), be1.reshape(1, HID),
                     g2.reshape(1, HID), be2.reshape(1, HID))


# speedup vs baseline: 1.2646x; 1.2646x over previous
import functools

import jax
import jax.numpy as jnp
from jax import lax
from jax.experimental import pallas as pl
from jax.experimental.pallas import tpu as pltpu
from jax.experimental.pallas import tpu_sc as plsc

N = 10000
E = 320000
HID = 128
NH = 8
HD = HID // NH
EPS = 1e-5

NC = 2
NS = 16
NW = NC * NS
EW = E // NW
C = 16
NCHUNK = EW // C
NP = 10240
NP8 = NP // 8
AROWS = NP + NP8
RPT = AROWS // NS
NB = NP // NW

_GDN = lax.GatherDimensionNumbers(
    offset_dims=(), collapsed_slice_dims=(0,), start_index_map=(0,))


def _perm(x, idx):
    return lax.gather(x, idx[:, None], _GDN, (1,),
                      mode=lax.GatherScatterMode.PROMISE_IN_BOUNDS)


def _qkv_body(h_ref, wq_ref, bq_ref, wk_ref, bk_ref, wv_ref, bv_ref,
              qs_ref, k_ref, v_ref):
    h = h_ref[...]
    scaling = HD ** (-0.5)
    qs_ref[...] = (jnp.dot(h, wq_ref[...],
                           preferred_element_type=jnp.float32)
                   + bq_ref[...]) * scaling
    k_ref[...] = jnp.dot(h, wk_ref[...],
                         preferred_element_type=jnp.float32) + bk_ref[...]
    v_ref[...] = jnp.dot(h, wv_ref[...],
                         preferred_element_type=jnp.float32) + bv_ref[...]


_qkv_call = pl.pallas_call(
    _qkv_body,
    out_shape=[
        jax.ShapeDtypeStruct((N, HID), jnp.float32),
        jax.ShapeDtypeStruct((N, HID), jnp.float32),
        jax.ShapeDtypeStruct((N, HID), jnp.float32),
    ],
)


def _sc_body(rows_hbm, cols_hbm, qs_hbm, k_hbm, v_hbm,
             out_hbm,
             acc,
             rows_a, cols_a, rows_b, cols_b,
             qb_a, kb_a, vb_a, ev_a, qb_b, kb_b, vb_b, ev_b, zq,
             sq_a, sk_a, sv_a, sm_a, se_a,
             sq_b, sk_b, sv_b, sm_b, se_b):
    c = lax.axis_index("c")
    s = lax.axis_index("s")
    wid = s * NC + c
    base = wid * EW
    lane = lax.iota(jnp.int32, 16)
    idx_hi = (lane + 8) & 15
    idx_mod8 = lane & 7
    zero16 = jnp.zeros((16,), jnp.float32)

    def _zq_row(i, _):
        for j in range(8):
            zq[i, pl.ds(16 * j, 16)] = zero16
        return 0
    lax.fori_loop(0, 16, _zq_row, 0)

    r0 = s * RPT

    def _zcopy(t, _):
        pltpu.sync_copy(zq, acc.at[pl.ds(r0 + t * 16, 16)])
        return 0
    lax.fori_loop(0, RPT // 16, _zcopy, 0)

    plsc.subcore_barrier()

    DEN_BATCHES = ((0, 0),)

    def _load_idx(ch, rv, cv):
        off = base + ch * C
        pltpu.sync_copy(rows_hbm.at[pl.ds(off, C)], rv)
        pltpu.sync_copy(cols_hbm.at[pl.ds(off, C)], cv)

    def _start_gathers(rv, cv, qb, kb, vb, sq, sk, sv):
        rr = rv[pl.ds(0, 16)]
        cc = cv[pl.ds(0, 16)]
        pltpu.async_copy(qs_hbm.at[rr], qb, sq)
        pltpu.async_copy(k_hbm.at[cc], kb, sk)
        pltpu.async_copy(v_hbm.at[cc], vb, sv)

    def _wait_gathers(rv, cv, qb, kb, vb, sq, sk, sv):
        pltpu.make_async_copy(qs_hbm.at[pl.ds(0, 16)], qb, sq).wait()
        pltpu.make_async_copy(k_hbm.at[pl.ds(0, 16)], kb, sk).wait()
        pltpu.make_async_copy(v_hbm.at[pl.ds(0, 16)], vb, sv).wait()

    def _compute(rv, qb, kb, vb, ev):
        k7 = (rv[pl.ds(0, 16)] & 7).astype(jnp.float32)

        def _edge(e, _):
            dot = qb[e, pl.ds(0, 16)] * kb[e, pl.ds(0, 16)]
            for j in range(1, 8):
                dot = dot + qb[e, pl.ds(16 * j, 16)] * kb[e, pl.ds(16 * j, 16)]
            s16 = dot + _perm(dot, idx_hi)
            s16 = jnp.where(lane < 8, s16, 0.0)
            e16 = jnp.exp(s16)
            e16z = jnp.where(lane < 8, e16, 0.0)
            ks = _perm(k7, lane * 0 + e)
            for j in range(8):
                d = ks - float(j)
                ev[e, pl.ds(16 * j, 16)] = e16z * jnp.maximum(1.0 - d * d, 0.0)
            pat = _perm(e16, idx_mod8)
            for j in range(8):
                vb[e, pl.ds(16 * j, 16)] = vb[e, pl.ds(16 * j, 16)] * pat
            return 0

        def _edge_pl(e):
            _edge(e, 0)
        plsc.parallel_loop(0, C, 1, unroll=2)(_edge_pl)

    def _start_scatters(rv, vb, ev, sm, se):
        rr = rv[pl.ds(0, 16)]
        di = NP + lax.shift_right_logical(rr, 3)
        pltpu.async_copy(vb, acc.at[rr], sm, add=True)
        pltpu.async_copy(ev, acc.at[di], se, add=True)

    def _drain_scatters(rv, vb, ev, sm, se):
        pltpu.make_async_copy(vb, acc.at[pl.ds(0, 16)], sm).wait()
        pltpu.make_async_copy(ev, acc.at[pl.ds(0, 16)], se).wait()

    _load_idx(0, rows_a, cols_a)
    _start_gathers(rows_a, cols_a, qb_a, kb_a, vb_a, sq_a, sk_a, sv_a)

    def _pair(i, _):
        @pl.when(i > 0)
        def _():
            _drain_scatters(rows_b, vb_b, ev_b, sm_b, se_b)
        _load_idx(2 * i + 1, rows_b, cols_b)
        _start_gathers(rows_b, cols_b, qb_b, kb_b, vb_b, sq_b, sk_b, sv_b)
        _wait_gathers(rows_a, cols_a, qb_a, kb_a, vb_a, sq_a, sk_a, sv_a)
        _compute(rows_a, qb_a, kb_a, vb_a, ev_a)
        _start_scatters(rows_a, vb_a, ev_a, sm_a, se_a)
        _drain_scatters(rows_a, vb_a, ev_a, sm_a, se_a)
        _load_idx(2 * i + 2, rows_a, cols_a)
        _start_gathers(rows_a, cols_a, qb_a, kb_a, vb_a, sq_a, sk_a, sv_a)
        _wait_gathers(rows_b, cols_b, qb_b, kb_b, vb_b, sq_b, sk_b, sv_b)
        _compute(rows_b, qb_b, kb_b, vb_b, ev_b)
        _start_scatters(rows_b, vb_b, ev_b, sm_b, se_b)
        return 0

    lax.fori_loop(0, NCHUNK // 2, _pair, 0)
    _drain_scatters(rows_b, vb_b, ev_b, sm_b, se_b)
    _wait_gathers(rows_a, cols_a, qb_a, kb_a, vb_a, sq_a, sk_a, sv_a)
    _compute(rows_a, qb_a, kb_a, vb_a, ev_a)
    _start_scatters(rows_a, vb_a, ev_a, sm_a, se_a)
    _drain_scatters(rows_a, vb_a, ev_a, sm_a, se_a)
    plsc.subcore_barrier()

    def _wcopy(t, _):
        pltpu.sync_copy(acc.at[pl.ds(r0 + t * 16, 16)],
                        out_hbm.at[c, pl.ds(r0 + t * 16, 16)])
        return 0
    lax.fori_loop(0, RPT // 16, _wcopy, 0)


_sc_call = functools.partial(
    pl.kernel,
    out_type=jax.ShapeDtypeStruct((NC, AROWS, HID), jnp.float32),
    mesh=plsc.VectorSubcoreMesh(
        core_axis_name="c", subcore_axis_name="s",
        num_cores=NC, num_subcores=NS),
    scratch_types=[
        pltpu.VMEM_SHARED((AROWS, HID), jnp.float32),
        pltpu.VMEM((C,), jnp.int32),
        pltpu.VMEM((C,), jnp.int32),
        pltpu.VMEM((C,), jnp.int32),
        pltpu.VMEM((C,), jnp.int32),
        pltpu.VMEM((C, HID), jnp.float32),
        pltpu.VMEM((C, HID), jnp.float32),
        pltpu.VMEM((C, HID), jnp.float32),
        pltpu.VMEM((C, HID), jnp.float32),
        pltpu.VMEM((C, HID), jnp.float32),
        pltpu.VMEM((C, HID), jnp.float32),
        pltpu.VMEM((C, HID), jnp.float32),
        pltpu.VMEM((C, HID), jnp.float32),
        pltpu.VMEM((16, HID), jnp.float32),
    ] + [pltpu.SemaphoreType.DMA] * 10,
)(_sc_body)


PC = 64


def _norm_body(all_hbm,
               att_hbm,
               o0, o1, d0, d1, obuf, sem0, sem1, sem2, sem3):
    c = lax.axis_index("c")
    s = lax.axis_index("s")
    wid = s * NC + c
    nb = wid * NB
    lane = lax.iota(jnp.int32, 16)
    idx_mod8 = lane & 7

    def _chunk(t, _):
        nb_t = pl.multiple_of(nb + t * PC, 64)
        dof = pl.multiple_of(NP + wid * (NB // 8) + t * (PC // 8), 8)
        cp0 = pltpu.async_copy(all_hbm.at[0, pl.ds(nb_t, PC)], o0, sem0)
        cp1 = pltpu.async_copy(all_hbm.at[1, pl.ds(nb_t, PC)], o1, sem1)
        cp2 = pltpu.async_copy(all_hbm.at[0, pl.ds(dof, PC // 8)], d0, sem2)
        cp3 = pltpu.async_copy(all_hbm.at[1, pl.ds(dof, PC // 8)], d1, sem3)
        cp0.wait()
        cp1.wait()
        cp2.wait()
        cp3.wait()

        def _drow(dr, _):
            for j in range(8):
                slot = d0[dr, pl.ds(16 * j, 16)] + d1[dr, pl.ds(16 * j, 16)]
                recip = 1.0 / jnp.maximum(slot, 1e-30)
                pat = _perm(recip, idx_mod8)
                i = dr * 8 + j
                for m in range(8):
                    obuf[i, pl.ds(16 * m, 16)] = (
                        o0[i, pl.ds(16 * m, 16)]
                        + o1[i, pl.ds(16 * m, 16)]) * pat
            return 0

        lax.fori_loop(0, PC // 8, _drow, 0)
        pltpu.sync_copy(obuf, att_hbm.at[pl.ds(nb_t, PC)])
        return 0

    lax.fori_loop(0, NB // PC, _chunk, 0)


_norm_call = functools.partial(
    pl.kernel,
    out_type=jax.ShapeDtypeStruct((NP, HID), jnp.float32),
    mesh=plsc.VectorSubcoreMesh(
        core_axis_name="c", subcore_axis_name="s",
        num_cores=NC, num_subcores=NS),
    scratch_types=[
        pltpu.VMEM((PC, HID), jnp.float32),
        pltpu.VMEM((PC, HID), jnp.float32),
        pltpu.VMEM((PC // 8, HID), jnp.float32),
        pltpu.VMEM((PC // 8, HID), jnp.float32),
        pltpu.VMEM((PC, HID), jnp.float32),
        pltpu.SemaphoreType.DMA,
        pltpu.SemaphoreType.DMA,
        pltpu.SemaphoreType.DMA,
        pltpu.SemaphoreType.DMA,
    ],
)(_norm_body)


def _gelu(x):
    return 0.5 * x * (1.0 + lax.erf(x * (2.0 ** -0.5)))


def _bn(x, gamma, beta):
    mean = jnp.mean(x, axis=0)
    var = jnp.mean((x - mean) ** 2, axis=0)
    return (x - mean) / jnp.sqrt(var + EPS) * gamma + beta


def _epi_body(a_ref, h_ref, wo_ref, bo_ref, w1_ref, b1_ref,
              w2_ref, b2_ref, g1_ref, be1_ref, g2_ref, be2_ref, out_ref):
    attnout = a_ref[:N]
    mha = jnp.dot(attnout, wo_ref[...],
                  preferred_element_type=jnp.float32) + bo_ref[...]
    x1 = _bn(mha + h_ref[...], g1_ref[...], be1_ref[...])
    h1 = _gelu(x1)
    ffn = _gelu(jnp.dot(h1, w1_ref[...],
                        preferred_element_type=jnp.float32) + b1_ref[...])
    ffn = jnp.dot(ffn, w2_ref[...],
                  preferred_element_type=jnp.float32) + b2_ref[...]
    out_ref[...] = _bn(h1 + ffn, g2_ref[...], be2_ref[...])


_epi_call = pl.pallas_call(
    _epi_body,
    out_shape=jax.ShapeDtypeStruct((N, HID), jnp.float32),
)


def kernel(edge_index, h, Wq, bq, Wk, bk, Wv, bv, Wo, bo, W1, b1, W2, b2,
           g1, be1, g2, be2):
    rows = edge_index[0]
    cols = edge_index[1]
    qs, kk, vv = _qkv_call(h, Wq, bq.reshape(1, HID), Wk, bk.reshape(1, HID),
                           Wv, bv.reshape(1, HID))
    out_all = _sc_call(rows, cols, qs, kk, vv)
    att = _norm_call(out_all)
    return _epi_call(att, h, Wo, bo.reshape(1, HID),
                     W1, b1.reshape(1, 2 * HID), W2, b2.reshape(1, HID),
                     g1.reshape(1, HID), be1.reshape(1, HID),
                     g2.reshape(1, HID), be2.reshape(1, HID))
